# baseline (speedup 1.0000x reference)
import jax
import jax.numpy as jnp
from jax.experimental import pallas as pl
from jax.experimental.pallas import tpu as pltpu

N = 4096
C = 32
BLOCK_ROWS = 512
NUM_BLOCKS = N // BLOCK_ROWS
NBUF = 3
TAIL_ROWS = 512
NUM_TAIL = N // TAIL_ROWS

F8 = jnp.float8_e4m3fn
LAP_SCALE = 256.0
DEF = jax.lax.Precision.DEFAULT


def _snn_body(x_ref, ld_ref, lu_ref, w_ref, out_ref,
              buf0, buf1, buf2, ldq, luq, u1f, v1f, u1q, v1q, zd,
              sem0, sem1, sem2):
    bufs = (buf0, buf1, buf2)
    sems = (sem0, sem1, sem2)
    mats = [ld_ref] * NUM_BLOCKS + [lu_ref] * NUM_BLOCKS
    n_steps = 2 * NUM_BLOCKS

    def fetch(s):
        blk = s % NUM_BLOCKS
        cp = pltpu.make_async_copy(
            mats[s].at[pl.ds(blk * BLOCK_ROWS, BLOCK_ROWS), :],
            bufs[s % NBUF], sems[s % NBUF])
        cp.start()
        return cp

    pending = {s: fetch(s) for s in range(NBUF - 1)}
    for s in range(n_steps):
        if s + NBUF - 1 < n_steps:
            pending[s + NBUF - 1] = fetch(s + NBUF - 1)
        blk = s % NUM_BLOCKS
        rows = pl.ds(blk * BLOCK_ROWS, BLOCK_ROWS)
        if s >= NUM_BLOCKS:
            zd[rows, :] = jnp.dot(ldq[rows, :], u1q[...],
                                  preferred_element_type=jnp.float32)
        pending.pop(s).wait()
        b = bufs[s % NBUF][...]
        r1 = jnp.dot(b, x_ref[...], precision=DEF,
                     preferred_element_type=jnp.float32)
        if s < NUM_BLOCKS:
            u1f[rows, :] = r1
            u1q[rows, :] = r1.astype(F8)
            ldq[rows, :] = (b * LAP_SCALE).astype(F8)
        else:
            v1f[rows, :] = r1
            v1q[rows, :] = r1.astype(F8)
            luq[rows, :] = (b * LAP_SCALE).astype(F8)

    w = w_ref[...]
    for j in range(NUM_TAIL):
        rows = pl.ds(j * TAIL_ROWS, TAIL_ROWS)
        zu = jnp.dot(luq[rows, :], v1q[...],
                     preferred_element_type=jnp.float32)
        cat = jnp.concatenate(
            [x_ref[rows, :], u1f[rows, :], zd[rows, :], v1f[rows, :], zu],
            axis=1)
        y = jnp.dot(cat, w, precision=DEF, preferred_element_type=jnp.float32)
        out_ref[rows, :] = jax.nn.sigmoid(y)


@jax.jit
def kernel(x, laplacian_down, laplacian_up, weight):
    wt = jnp.transpose(weight, (2, 0, 1)).reshape(5 * C, C)
    scale = jnp.ones((5, 1, 1), jnp.float32).at[2].set(1.0 / LAP_SCALE)
    scale = scale.at[4].set(1.0 / LAP_SCALE)
    wt = (wt.reshape(5, C, C) * scale).reshape(5 * C, C)
    return pl.pallas_call(
        _snn_body,
        grid=(1,),
        in_specs=[
            pl.BlockSpec((N, C), lambda s: (0, 0)),
            pl.BlockSpec(memory_space=pl.ANY),
            pl.BlockSpec(memory_space=pl.ANY),
            pl.BlockSpec((5 * C, C), lambda s: (0, 0)),
        ],
        out_specs=pl.BlockSpec((N, C), lambda s: (0, 0)),
        out_shape=jax.ShapeDtypeStruct((N, C), jnp.float32),
        scratch_shapes=[
            pltpu.VMEM((BLOCK_ROWS, N), jnp.float32),
            pltpu.VMEM((BLOCK_ROWS, N), jnp.float32),
            pltpu.VMEM((BLOCK_ROWS, N), jnp.float32),
            pltpu.VMEM((N, N), F8),
            pltpu.VMEM((N, N), F8),
            pltpu.VMEM((N, C), jnp.float32),
            pltpu.VMEM((N, C), jnp.float32),
            pltpu.VMEM((N, C), F8),
            pltpu.VMEM((N, C), F8),
            pltpu.VMEM((N, C), jnp.float32),
            pltpu.SemaphoreType.DMA,
            pltpu.SemaphoreType.DMA,
            pltpu.SemaphoreType.DMA,
        ],
    )(x, laplacian_down, laplacian_up, wt)


# speedup vs baseline: 1.0576x; 1.0576x over previous
import jax
import jax.numpy as jnp
from jax.experimental import pallas as pl
from jax.experimental.pallas import tpu as pltpu

N = 4096
C = 32
BLOCK_ROWS = 128
NUM_BLOCKS = N // BLOCK_ROWS
NBUF = 8
TAIL_ROWS = 512
NUM_TAIL = N // TAIL_ROWS

F8 = jnp.float8_e4m3fn
LAP_SCALE = 256.0
DEF = jax.lax.Precision.DEFAULT


def _snn_body(x_ref, ld_ref, lu_ref, w_ref, out_ref,
              buf0, buf1, buf2, buf3, buf4, buf5, buf6, buf7,
              ldq, luq, u1f, v1f, u1q, v1q, zd,
              sem0, sem1, sem2, sem3, sem4, sem5, sem6, sem7):
    bufs = (buf0, buf1, buf2, buf3, buf4, buf5, buf6, buf7)
    sems = (sem0, sem1, sem2, sem3, sem4, sem5, sem6, sem7)
    mats = [ld_ref] * NUM_BLOCKS + [lu_ref] * NUM_BLOCKS
    n_steps = 2 * NUM_BLOCKS

    def fetch(s):
        blk = s % NUM_BLOCKS
        cp = pltpu.make_async_copy(
            mats[s].at[pl.ds(blk * BLOCK_ROWS, BLOCK_ROWS), :],
            bufs[s % NBUF], sems[s % NBUF])
        cp.start()
        return cp

    pending = {s: fetch(s) for s in range(NBUF - 1)}
    for s in range(n_steps):
        if s + NBUF - 1 < n_steps:
            pending[s + NBUF - 1] = fetch(s + NBUF - 1)
        blk = s % NUM_BLOCKS
        rows = pl.ds(blk * BLOCK_ROWS, BLOCK_ROWS)
        if s >= NUM_BLOCKS:
            zd[rows, :] = jnp.dot(ldq[rows, :], u1q[...],
                                  preferred_element_type=jnp.float32)
        pending.pop(s).wait()
        b = bufs[s % NBUF][...]
        r1 = jnp.dot(b, x_ref[...], precision=DEF,
                     preferred_element_type=jnp.float32)
        if s < NUM_BLOCKS:
            u1f[rows, :] = r1
            u1q[rows, :] = r1.astype(F8)
            ldq[rows, :] = (b * LAP_SCALE).astype(F8)
        else:
            v1f[rows, :] = r1
            v1q[rows, :] = r1.astype(F8)
            luq[rows, :] = (b * LAP_SCALE).astype(F8)

    w = w_ref[...]
    for j in range(NUM_TAIL):
        rows = pl.ds(j * TAIL_ROWS, TAIL_ROWS)
        zu = jnp.dot(luq[rows, :], v1q[...],
                     preferred_element_type=jnp.float32)
        cat = jnp.concatenate(
            [x_ref[rows, :], u1f[rows, :], zd[rows, :], v1f[rows, :], zu],
            axis=1)
        y = jnp.dot(cat, w, precision=DEF, preferred_element_type=jnp.float32)
        out_ref[rows, :] = jax.nn.sigmoid(y)


@jax.jit
def kernel(x, laplacian_down, laplacian_up, weight):
    wt = jnp.transpose(weight, (2, 0, 1)).reshape(5 * C, C)
    scale = jnp.ones((5, 1, 1), jnp.float32).at[2].set(1.0 / LAP_SCALE)
    scale = scale.at[4].set(1.0 / LAP_SCALE)
    wt = (wt.reshape(5, C, C) * scale).reshape(5 * C, C)
    return pl.pallas_call(
        _snn_body,
        grid=(1,),
        in_specs=[
            pl.BlockSpec((N, C), lambda s: (0, 0)),
            pl.BlockSpec(memory_space=pl.ANY),
            pl.BlockSpec(memory_space=pl.ANY),
            pl.BlockSpec((5 * C, C), lambda s: (0, 0)),
        ],
        out_specs=pl.BlockSpec((N, C), lambda s: (0, 0)),
        out_shape=jax.ShapeDtypeStruct((N, C), jnp.float32),
        scratch_shapes=[
            pltpu.VMEM((BLOCK_ROWS, N), jnp.float32),
            pltpu.VMEM((BLOCK_ROWS, N), jnp.float32),
            pltpu.VMEM((BLOCK_ROWS, N), jnp.float32),
            pltpu.VMEM((BLOCK_ROWS, N), jnp.float32),
            pltpu.VMEM((BLOCK_ROWS, N), jnp.float32),
            pltpu.VMEM((BLOCK_ROWS, N), jnp.float32),
            pltpu.VMEM((BLOCK_ROWS, N), jnp.float32),
            pltpu.VMEM((BLOCK_ROWS, N), jnp.float32),
            pltpu.VMEM((N, N), F8),
            pltpu.VMEM((N, N), F8),
            pltpu.VMEM((N, C), jnp.float32),
            pltpu.VMEM((N, C), jnp.float32),
            pltpu.VMEM((N, C), F8),
            pltpu.VMEM((N, C), F8),
            pltpu.VMEM((N, C), jnp.float32),
            pltpu.SemaphoreType.DMA,
            pltpu.SemaphoreType.DMA,
            pltpu.SemaphoreType.DMA,
            pltpu.SemaphoreType.DMA,
            pltpu.SemaphoreType.DMA,
            pltpu.SemaphoreType.DMA,
            pltpu.SemaphoreType.DMA,
            pltpu.SemaphoreType.DMA,
        ],
    )(x, laplacian_down, laplacian_up, wt)


# speedup vs baseline: 1.0788x; 1.0200x over previous
import jax
import jax.numpy as jnp
from jax.experimental import pallas as pl
from jax.experimental.pallas import tpu as pltpu

N = 4096
C = 32
BLOCK_ROWS = 256
NUM_BLOCKS = N // BLOCK_ROWS
NBUF = 5
TAIL_ROWS = 512
NUM_TAIL = N // TAIL_ROWS

F8 = jnp.float8_e4m3fn
LAP_SCALE = 256.0
DEF = jax.lax.Precision.DEFAULT


def _snn_body(x_ref, ld_ref, lu_ref, w_ref, out_ref,
              buf0, buf1, buf2, buf3, buf4, ldq, luq, u1f, v1f, u1q, v1q, zd,
              sem0, sem1, sem2, sem3, sem4):
    bufs = (buf0, buf1, buf2, buf3, buf4)
    sems = (sem0, sem1, sem2, sem3, sem4)
    mats = [ld_ref] * NUM_BLOCKS + [lu_ref] * NUM_BLOCKS
    n_steps = 2 * NUM_BLOCKS

    def fetch(s):
        blk = s % NUM_BLOCKS
        cp = pltpu.make_async_copy(
            mats[s].at[pl.ds(blk * BLOCK_ROWS, BLOCK_ROWS), :],
            bufs[s % NBUF], sems[s % NBUF])
        cp.start()
        return cp

    pending = {s: fetch(s) for s in range(NBUF - 1)}
    for s in range(n_steps):
        if s + NBUF - 1 < n_steps:
            pending[s + NBUF - 1] = fetch(s + NBUF - 1)
        blk = s % NUM_BLOCKS
        rows = pl.ds(blk * BLOCK_ROWS, BLOCK_ROWS)
        if s >= NUM_BLOCKS:
            zd[rows, :] = jnp.dot(ldq[rows, :], u1q[...],
                                  preferred_element_type=jnp.float32)
        pending.pop(s).wait()
        b = bufs[s % NBUF][...]
        r1 = jnp.dot(b, x_ref[...], precision=DEF,
                     preferred_element_type=jnp.float32)
        if s < NUM_BLOCKS:
            u1f[rows, :] = r1
            u1q[rows, :] = r1.astype(F8)
            ldq[rows, :] = (b * LAP_SCALE).astype(F8)
        else:
            v1f[rows, :] = r1
            v1q[rows, :] = r1.astype(F8)
            luq[rows, :] = (b * LAP_SCALE).astype(F8)

    w = w_ref[...]
    for j in range(NUM_TAIL):
        rows = pl.ds(j * TAIL_ROWS, TAIL_ROWS)
        zu = jnp.dot(luq[rows, :], v1q[...],
                     preferred_element_type=jnp.float32)
        cat = jnp.concatenate(
            [x_ref[rows, :], u1f[rows, :], zd[rows, :], v1f[rows, :], zu],
            axis=1)
        y = jnp.dot(cat, w, precision=DEF, preferred_element_type=jnp.float32)
        out_ref[rows, :] = jax.nn.sigmoid(y)


@jax.jit
def kernel(x, laplacian_down, laplacian_up, weight):
    wt = jnp.transpose(weight, (2, 0, 1)).reshape(5 * C, C)
    scale = jnp.ones((5, 1, 1), jnp.float32).at[2].set(1.0 / LAP_SCALE)
    scale = scale.at[4].set(1.0 / LAP_SCALE)
    wt = (wt.reshape(5, C, C) * scale).reshape(5 * C, C)
    return pl.pallas_call(
        _snn_body,
        grid=(1,),
        in_specs=[
            pl.BlockSpec((N, C), lambda s: (0, 0)),
            pl.BlockSpec(memory_space=pl.ANY),
            pl.BlockSpec(memory_space=pl.ANY),
            pl.BlockSpec((5 * C, C), lambda s: (0, 0)),
        ],
        out_specs=pl.BlockSpec((N, C), lambda s: (0, 0)),
        out_shape=jax.ShapeDtypeStruct((N, C), jnp.float32),
        scratch_shapes=[
            pltpu.VMEM((BLOCK_ROWS, N), jnp.float32),
            pltpu.VMEM((BLOCK_ROWS, N), jnp.float32),
            pltpu.VMEM((BLOCK_ROWS, N), jnp.float32),
            pltpu.VMEM((BLOCK_ROWS, N), jnp.float32),
            pltpu.VMEM((BLOCK_ROWS, N), jnp.float32),
            pltpu.VMEM((N, N), F8),
            pltpu.VMEM((N, N), F8),
            pltpu.VMEM((N, C), jnp.float32),
            pltpu.VMEM((N, C), jnp.float32),
            pltpu.VMEM((N, C), F8),
            pltpu.VMEM((N, C), F8),
            pltpu.VMEM((N, C), jnp.float32),
            pltpu.SemaphoreType.DMA,
            pltpu.SemaphoreType.DMA,
            pltpu.SemaphoreType.DMA,
            pltpu.SemaphoreType.DMA,
            pltpu.SemaphoreType.DMA,
        ],
    )(x, laplacian_down, laplacian_up, wt)
